# Initial kernel scaffold; baseline (speedup 1.0000x reference)
import jax
import jax.numpy as jnp
from jax.experimental import pallas as pl


def kernel(h, edge_index, edge_weight, We, be, Wg, W_ih, W_hh, b_ih, b_hh, Wm, bm):
    raise NotImplementedError("write your pallas kernel here")


# speedup vs baseline: 4.0596x; 4.0596x over previous
import functools

import jax
import jax.numpy as jnp
from jax import lax
from jax.experimental import pallas as pl
from jax.experimental.pallas import tpu as pltpu
from jax.experimental.pallas import tpu_sc as plsc

_NC = 2
_NS = 16
_NW = _NC * _NS


def _sc_spmm(m, src, dst, w):
    N, H = m.shape
    E = src.shape[0]
    EPW = E // _NW
    CH = 80
    NCH = EPW // CH
    RPS = 624
    TAIL = N - _NS * RPS
    TBASE = _NS * RPS

    mesh = plsc.VectorSubcoreMesh(
        core_axis_name="c", subcore_axis_name="s",
        num_cores=_NC, num_subcores=_NS)

    @functools.partial(
        pl.kernel,
        out_type=jax.ShapeDtypeStruct((_NC, N, H), jnp.float32),
        mesh=mesh,
        scratch_types=[
            pltpu.VMEM((CH,), jnp.int32),
            pltpu.VMEM((CH,), jnp.int32),
            pltpu.VMEM((CH,), jnp.float32),
            pltpu.VMEM((CH, H), jnp.float32),
            pltpu.VMEM_SHARED((N, H), jnp.float32),
            pltpu.SemaphoreType.DMA,
        ],
    )
    def k(m_hbm, src_hbm, dst_hbm, w_hbm, out_hbm,
          src_v, dst_v, w_v, rows_v, agg_sh, sem):
        cid = lax.axis_index("c")
        sid = lax.axis_index("s")
        wid = cid * _NS + sid

        zero16 = jnp.zeros((16,), jnp.float32)

        def zrow(i, carry):
            for j in range(H // 16):
                rows_v[i, pl.ds(j * 16, 16)] = zero16
            return carry
        lax.fori_loop(0, CH, zrow, 0)
        rbase = sid * RPS
        for t in range(RPS // CH):
            pltpu.sync_copy(rows_v, agg_sh.at[pl.ds(rbase + t * CH, CH)])
        REM = RPS % CH
        if REM:
            pltpu.sync_copy(rows_v.at[pl.ds(0, REM)],
                            agg_sh.at[pl.ds(rbase + RPS - REM, REM)])

        @pl.when(sid == _NS - 1)
        def _zero_tail():
            pltpu.sync_copy(rows_v.at[pl.ds(0, TAIL)],
                            agg_sh.at[pl.ds(TBASE, TAIL)])
        plsc.subcore_barrier()

        ebase = wid * EPW

        def chunk(ci, carry):
            off = ebase + ci * CH
            pltpu.sync_copy(src_hbm.at[pl.ds(off, CH)], src_v)
            pltpu.sync_copy(dst_hbm.at[pl.ds(off, CH)], dst_v)
            pltpu.sync_copy(w_hbm.at[pl.ds(off, CH)], w_v)
            pltpu.async_copy(m_hbm.at[src_v], rows_v, sem).wait()

            def scale(g, c2):
                base = g * 16
                w16 = w_v[pl.ds(base, 16)]
                for r in range(16):
                    wv = jnp.full((16,), w16[r])
                    for j in range(H // 16):
                        rows_v[base + r, pl.ds(j * 16, 16)] = (
                            rows_v[base + r, pl.ds(j * 16, 16)] * wv)
                return c2
            lax.fori_loop(0, CH // 16, scale, 0)

            pltpu.sync_copy(rows_v, agg_sh.at[dst_v], add=True)
            return carry
        lax.fori_loop(0, NCH, chunk, 0)

        plsc.subcore_barrier()
        for t in range(RPS // CH):
            pltpu.sync_copy(agg_sh.at[pl.ds(rbase + t * CH, CH)], rows_v)
            pltpu.sync_copy(rows_v, out_hbm.at[cid, pl.ds(rbase + t * CH, CH)])
        if REM:
            pltpu.sync_copy(agg_sh.at[pl.ds(rbase + RPS - REM, REM)],
                            rows_v.at[pl.ds(0, REM)])
            pltpu.sync_copy(rows_v.at[pl.ds(0, REM)],
                            out_hbm.at[cid, pl.ds(rbase + RPS - REM, REM)])

        @pl.when(sid == _NS - 1)
        def _drain_tail():
            pltpu.sync_copy(agg_sh.at[pl.ds(TBASE, TAIL)],
                            rows_v.at[pl.ds(0, TAIL)])
            pltpu.sync_copy(rows_v.at[pl.ds(0, TAIL)],
                            out_hbm.at[cid, pl.ds(TBASE, TAIL)])

    return k(m, src, dst, w)


_BN = 1000


def _embed_body(h_ref, WeT_ref, be_ref, Wg0_ref, x_ref, m0_ref):
    x = jnp.dot(h_ref[...], WeT_ref[...],
                preferred_element_type=jnp.float32) + be_ref[...]
    x_ref[...] = x
    m0_ref[...] = jnp.dot(x, Wg0_ref[...], preferred_element_type=jnp.float32)


def _embed(h, WeT, be2, Wg0):
    N, D = h.shape
    H = WeT.shape[1]
    grid = (N // _BN,)
    return pl.pallas_call(
        _embed_body,
        grid=grid,
        in_specs=[
            pl.BlockSpec((_BN, D), lambda i: (i, 0)),
            pl.BlockSpec((D, H), lambda i: (0, 0)),
            pl.BlockSpec((1, H), lambda i: (0, 0)),
            pl.BlockSpec((H, H), lambda i: (0, 0)),
        ],
        out_specs=[
            pl.BlockSpec((_BN, H), lambda i: (i, 0)),
            pl.BlockSpec((_BN, H), lambda i: (i, 0)),
        ],
        out_shape=[
            jax.ShapeDtypeStruct((N, H), jnp.float32),
            jax.ShapeDtypeStruct((N, H), jnp.float32),
        ],
    )(h, WeT, be2, Wg0)


def _gru_body(agg2_ref, x_ref, WihT_ref, WhhT_ref, bih_ref, bhh_ref, Wgn_ref,
              xn_ref, mn_ref):
    H = x_ref.shape[1]
    agg = agg2_ref[0] + agg2_ref[1]
    x = x_ref[...]
    gi = jnp.dot(agg, WihT_ref[...],
                 preferred_element_type=jnp.float32) + bih_ref[...]
    gh = jnp.dot(x, WhhT_ref[...],
                 preferred_element_type=jnp.float32) + bhh_ref[...]
    r = jax.nn.sigmoid(gi[:, :H] + gh[:, :H])
    z = jax.nn.sigmoid(gi[:, H:2 * H] + gh[:, H:2 * H])
    n = jnp.tanh(gi[:, 2 * H:] + r * gh[:, 2 * H:])
    xn = (1.0 - z) * n + z * x
    xn_ref[...] = xn
    mn_ref[...] = jnp.dot(xn, Wgn_ref[...], preferred_element_type=jnp.float32)


def _gru(agg2, x, WihT, WhhT, bih2, bhh2, Wgn):
    N, H = x.shape
    grid = (N // _BN,)
    return pl.pallas_call(
        _gru_body,
        grid=grid,
        in_specs=[
            pl.BlockSpec((_NC, _BN, H), lambda i: (0, i, 0)),
            pl.BlockSpec((_BN, H), lambda i: (i, 0)),
            pl.BlockSpec((H, 3 * H), lambda i: (0, 0)),
            pl.BlockSpec((H, 3 * H), lambda i: (0, 0)),
            pl.BlockSpec((1, 3 * H), lambda i: (0, 0)),
            pl.BlockSpec((1, 3 * H), lambda i: (0, 0)),
            pl.BlockSpec((H, H), lambda i: (0, 0)),
        ],
        out_specs=[
            pl.BlockSpec((_BN, H), lambda i: (i, 0)),
            pl.BlockSpec((_BN, H), lambda i: (i, 0)),
        ],
        out_shape=[
            jax.ShapeDtypeStruct((N, H), jnp.float32),
            jax.ShapeDtypeStruct((N, H), jnp.float32),
        ],
    )(agg2, x, WihT, WhhT, bih2, bhh2, Wgn)


def _mlp_body(xin_ref, x_ref, WmT_ref, bm_ref, out_ref):
    s = xin_ref[...] + x_ref[...]
    out_ref[...] = jnp.dot(s, WmT_ref[...],
                           preferred_element_type=jnp.float32) + bm_ref[...]


def _mlp(x_in, x, WmT_pad, bm_pad):
    N, H = x.shape
    CP = WmT_pad.shape[1]
    grid = (N // _BN,)
    return pl.pallas_call(
        _mlp_body,
        grid=grid,
        in_specs=[
            pl.BlockSpec((_BN, H), lambda i: (i, 0)),
            pl.BlockSpec((_BN, H), lambda i: (i, 0)),
            pl.BlockSpec((H, CP), lambda i: (0, 0)),
            pl.BlockSpec((1, CP), lambda i: (0, 0)),
        ],
        out_specs=pl.BlockSpec((_BN, CP), lambda i: (i, 0)),
        out_shape=jax.ShapeDtypeStruct((N, CP), jnp.float32),
    )(x_in, x, WmT_pad, bm_pad)


def kernel(h, edge_index, edge_weight, We, be, Wg, W_ih, W_hh, b_ih, b_hh,
           Wm, bm):
    L, H, _ = Wg.shape
    C = Wm.shape[0]
    src = edge_index[0].astype(jnp.int32)
    dst = edge_index[1].astype(jnp.int32)
    w = edge_weight.astype(jnp.float32)

    WeT = We.T
    be2 = be.reshape(1, H)
    WihT = W_ih.T
    WhhT = W_hh.T
    bih2 = b_ih.reshape(1, 3 * H)
    bhh2 = b_hh.reshape(1, 3 * H)
    CP = 128
    WmT_pad = jnp.zeros((H, CP), jnp.float32).at[:, :C].set(Wm.T)
    bm_pad = jnp.zeros((1, CP), jnp.float32).at[:, :C].set(bm.reshape(1, C))

    x, m = _embed(h, WeT, be2, Wg[0])
    x_in = x
    for i in range(L):
        agg2 = _sc_spmm(m, src, dst, w)
        Wgn = Wg[i + 1] if i + 1 < L else Wg[0]
        x, m = _gru(agg2, x, WihT, WhhT, bih2, bhh2, Wgn)
    out = _mlp(x_in, x, WmT_pad, bm_pad)
    return out[:, :C]


# speedup vs baseline: 6.9813x; 1.7197x over previous
import functools

import jax
import jax.numpy as jnp
from jax import lax
from jax.experimental import pallas as pl
from jax.experimental.pallas import tpu as pltpu
from jax.experimental.pallas import tpu_sc as plsc

_NC = 2
_NS = 16
_NW = _NC * _NS


_CH = 80
_HALVES = 2
_NCHH = 63
_HEPW = _CH * _NCHH
_EPWP = _HALVES * _HEPW


def _sc_spmm(m, src, dst, w):
    N, H = m.shape
    RPS = 624
    TAIL = N - _NS * RPS
    TBASE = _NS * RPS

    mesh = plsc.VectorSubcoreMesh(
        core_axis_name="c", subcore_axis_name="s",
        num_cores=_NC, num_subcores=_NS)

    @functools.partial(
        pl.kernel,
        out_type=jax.ShapeDtypeStruct((_NC, N, H), jnp.float32),
        mesh=mesh,
        scratch_types=[
            pltpu.VMEM((_HEPW,), jnp.int32),
            pltpu.VMEM((_HEPW,), jnp.int32),
            pltpu.VMEM((_HEPW,), jnp.float32),
            [pltpu.VMEM((_CH, H), jnp.float32) for _ in range(3)],
            [pltpu.VMEM((_CH,), jnp.int32) for _ in range(3)],
            pltpu.VMEM_SHARED((N, H), jnp.float32),
            [pltpu.SemaphoreType.DMA for _ in range(3)],
            [pltpu.SemaphoreType.DMA for _ in range(3)],
        ],
    )
    def k(m_hbm, src_hbm, dst_hbm, w_hbm, out_hbm,
          src_b, dst_b, w_b, rows, dst_v, agg_sh, gsem, ssem):
        cid = lax.axis_index("c")
        sid = lax.axis_index("s")
        wid = cid * _NS + sid

        zero16 = jnp.zeros((16,), jnp.float32)

        def zrow(i, carry):
            for j in range(H // 16):
                rows[0][i, pl.ds(j * 16, 16)] = zero16
            return carry
        lax.fori_loop(0, _CH, zrow, 0)
        rbase = sid * RPS
        for t in range(RPS // _CH):
            pltpu.sync_copy(rows[0], agg_sh.at[pl.ds(rbase + t * _CH, _CH)])
        REM = RPS % _CH
        if REM:
            pltpu.sync_copy(rows[0].at[pl.ds(0, REM)],
                            agg_sh.at[pl.ds(rbase + RPS - REM, REM)])

        @pl.when(sid == _NS - 1)
        def _zero_tail():
            pltpu.sync_copy(rows[0].at[pl.ds(0, TAIL)],
                            agg_sh.at[pl.ds(TBASE, TAIL)])
        plsc.subcore_barrier()

        def start_gather(c, k_):
            pltpu.async_copy(
                m_hbm.at[src_b.at[pl.ds(c * _CH, _CH)]], rows[k_], gsem[k_])

        def wait_gather(k_):
            pltpu.make_async_copy(
                m_hbm.at[src_b.at[pl.ds(0, _CH)]], rows[k_], gsem[k_]).wait()

        def start_scatter(k_):
            pltpu.async_copy(rows[k_], agg_sh.at[dst_v[k_]], ssem[k_],
                             add=True)

        def wait_scatter(k_):
            pltpu.make_async_copy(rows[k_], agg_sh.at[dst_v[k_]],
                                  ssem[k_]).wait()

        for h in range(_HALVES):
            base = wid * _EPWP + h * _HEPW
            pltpu.sync_copy(src_hbm.at[pl.ds(base, _HEPW)], src_b)
            pltpu.sync_copy(dst_hbm.at[pl.ds(base, _HEPW)], dst_b)
            pltpu.sync_copy(w_hbm.at[pl.ds(base, _HEPW)], w_b)
            start_gather(0, 0)

            def triple(t, carry):
                for k_ in range(3):
                    c = 3 * t + k_
                    kn = (k_ + 1) % 3

                    @pl.when(c >= 2)
                    def _w():
                        wait_scatter(kn)

                    @pl.when(c + 1 < _NCHH)
                    def _g():
                        start_gather(c + 1, kn)
                    wait_gather(k_)
                    for g in range(_CH // 16):
                        dst_v[k_][pl.ds(g * 16, 16)] = (
                            dst_b[pl.ds(c * _CH + g * 16, 16)])

                    def scale(g, c2):
                        w16 = w_b[pl.ds(c * _CH + g * 16, 16)]
                        for r in range(16):
                            wv = jnp.full((16,), w16[r])
                            for j in range(H // 16):
                                rows[k_][g * 16 + r, pl.ds(j * 16, 16)] = (
                                    rows[k_][g * 16 + r, pl.ds(j * 16, 16)]
                                    * wv)
                        return c2
                    lax.fori_loop(0, _CH // 16, scale, 0)
                    start_scatter(k_)
                return carry
            lax.fori_loop(0, _NCHH // 3, triple, 0)
            wait_scatter((_NCHH - 2) % 3)
            wait_scatter((_NCHH - 1) % 3)

        plsc.subcore_barrier()
        for t in range(RPS // _CH):
            pltpu.sync_copy(agg_sh.at[pl.ds(rbase + t * _CH, _CH)], rows[0])
            pltpu.sync_copy(rows[0],
                            out_hbm.at[cid, pl.ds(rbase + t * _CH, _CH)])
        if REM:
            pltpu.sync_copy(agg_sh.at[pl.ds(rbase + RPS - REM, REM)],
                            rows[0].at[pl.ds(0, REM)])
            pltpu.sync_copy(rows[0].at[pl.ds(0, REM)],
                            out_hbm.at[cid, pl.ds(rbase + RPS - REM, REM)])

        @pl.when(sid == _NS - 1)
        def _drain_tail():
            pltpu.sync_copy(agg_sh.at[pl.ds(TBASE, TAIL)],
                            rows[0].at[pl.ds(0, TAIL)])
            pltpu.sync_copy(rows[0].at[pl.ds(0, TAIL)],
                            out_hbm.at[cid, pl.ds(TBASE, TAIL)])

    return k(m, src, dst, w)


_BN = 1000


def _embed_body(h_ref, WeT_ref, be_ref, Wg0_ref, x_ref, m0_ref):
    x = jnp.dot(h_ref[...], WeT_ref[...],
                preferred_element_type=jnp.float32) + be_ref[...]
    x_ref[...] = x
    m0_ref[...] = jnp.dot(x, Wg0_ref[...], preferred_element_type=jnp.float32)


def _embed(h, WeT, be2, Wg0):
    N, D = h.shape
    H = WeT.shape[1]
    grid = (N // _BN,)
    return pl.pallas_call(
        _embed_body,
        grid=grid,
        in_specs=[
            pl.BlockSpec((_BN, D), lambda i: (i, 0)),
            pl.BlockSpec((D, H), lambda i: (0, 0)),
            pl.BlockSpec((1, H), lambda i: (0, 0)),
            pl.BlockSpec((H, H), lambda i: (0, 0)),
        ],
        out_specs=[
            pl.BlockSpec((_BN, H), lambda i: (i, 0)),
            pl.BlockSpec((_BN, H), lambda i: (i, 0)),
        ],
        out_shape=[
            jax.ShapeDtypeStruct((N, H), jnp.float32),
            jax.ShapeDtypeStruct((N, H), jnp.float32),
        ],
    )(h, WeT, be2, Wg0)


def _gru_body(agg2_ref, x_ref, WihT_ref, WhhT_ref, bih_ref, bhh_ref, Wgn_ref,
              xn_ref, mn_ref):
    H = x_ref.shape[1]
    agg = agg2_ref[0] + agg2_ref[1]
    x = x_ref[...]
    gi = jnp.dot(agg, WihT_ref[...],
                 preferred_element_type=jnp.float32) + bih_ref[...]
    gh = jnp.dot(x, WhhT_ref[...],
                 preferred_element_type=jnp.float32) + bhh_ref[...]
    r = jax.nn.sigmoid(gi[:, :H] + gh[:, :H])
    z = jax.nn.sigmoid(gi[:, H:2 * H] + gh[:, H:2 * H])
    n = jnp.tanh(gi[:, 2 * H:] + r * gh[:, 2 * H:])
    xn = (1.0 - z) * n + z * x
    xn_ref[...] = xn
    mn_ref[...] = jnp.dot(xn, Wgn_ref[...], preferred_element_type=jnp.float32)


def _gru(agg2, x, WihT, WhhT, bih2, bhh2, Wgn):
    N, H = x.shape
    grid = (N // _BN,)
    return pl.pallas_call(
        _gru_body,
        grid=grid,
        in_specs=[
            pl.BlockSpec((_NC, _BN, H), lambda i: (0, i, 0)),
            pl.BlockSpec((_BN, H), lambda i: (i, 0)),
            pl.BlockSpec((H, 3 * H), lambda i: (0, 0)),
            pl.BlockSpec((H, 3 * H), lambda i: (0, 0)),
            pl.BlockSpec((1, 3 * H), lambda i: (0, 0)),
            pl.BlockSpec((1, 3 * H), lambda i: (0, 0)),
            pl.BlockSpec((H, H), lambda i: (0, 0)),
        ],
        out_specs=[
            pl.BlockSpec((_BN, H), lambda i: (i, 0)),
            pl.BlockSpec((_BN, H), lambda i: (i, 0)),
        ],
        out_shape=[
            jax.ShapeDtypeStruct((N, H), jnp.float32),
            jax.ShapeDtypeStruct((N, H), jnp.float32),
        ],
    )(agg2, x, WihT, WhhT, bih2, bhh2, Wgn)


def _mlp_body(xin_ref, x_ref, WmT_ref, bm_ref, out_ref):
    s = xin_ref[...] + x_ref[...]
    out_ref[...] = jnp.dot(s, WmT_ref[...],
                           preferred_element_type=jnp.float32) + bm_ref[...]


def _mlp(x_in, x, WmT_pad, bm_pad):
    N, H = x.shape
    CP = WmT_pad.shape[1]
    grid = (N // _BN,)
    return pl.pallas_call(
        _mlp_body,
        grid=grid,
        in_specs=[
            pl.BlockSpec((_BN, H), lambda i: (i, 0)),
            pl.BlockSpec((_BN, H), lambda i: (i, 0)),
            pl.BlockSpec((H, CP), lambda i: (0, 0)),
            pl.BlockSpec((1, CP), lambda i: (0, 0)),
        ],
        out_specs=pl.BlockSpec((_BN, CP), lambda i: (i, 0)),
        out_shape=jax.ShapeDtypeStruct((N, CP), jnp.float32),
    )(x_in, x, WmT_pad, bm_pad)


def kernel(h, edge_index, edge_weight, We, be, Wg, W_ih, W_hh, b_ih, b_hh,
           Wm, bm):
    L, H, _ = Wg.shape
    C = Wm.shape[0]
    src = edge_index[0].astype(jnp.int32)
    dst = edge_index[1].astype(jnp.int32)
    w = edge_weight.astype(jnp.float32)
    E = src.shape[0]
    EPW = E // _NW
    PAD = _EPWP - EPW
    src = jnp.pad(src.reshape(_NW, EPW), ((0, 0), (0, PAD))).reshape(-1)
    dst = jnp.pad(dst.reshape(_NW, EPW), ((0, 0), (0, PAD))).reshape(-1)
    w = jnp.pad(w.reshape(_NW, EPW), ((0, 0), (0, PAD))).reshape(-1)

    WeT = We.T
    be2 = be.reshape(1, H)
    WihT = W_ih.T
    WhhT = W_hh.T
    bih2 = b_ih.reshape(1, 3 * H)
    bhh2 = b_hh.reshape(1, 3 * H)
    CP = 128
    WmT_pad = jnp.zeros((H, CP), jnp.float32).at[:, :C].set(Wm.T)
    bm_pad = jnp.zeros((1, CP), jnp.float32).at[:, :C].set(bm.reshape(1, C))

    x, m = _embed(h, WeT, be2, Wg[0])
    x_in = x
    for i in range(L):
        agg2 = _sc_spmm(m, src, dst, w)
        Wgn = Wg[i + 1] if i + 1 < L else Wg[0]
        x, m = _gru(agg2, x, WihT, WhhT, bih2, bhh2, Wgn)
    out = _mlp(x_in, x, WmT_pad, bm_pad)
    return out[:, :C]


# speedup vs baseline: 6.9858x; 1.0006x over previous
import functools

import jax
import jax.numpy as jnp
from jax import lax
from jax.experimental import pallas as pl
from jax.experimental.pallas import tpu as pltpu
from jax.experimental.pallas import tpu_sc as plsc

_NC = 2
_NS = 16
_NW = _NC * _NS


_CH = 80
_HALVES = 2
_NCHH = 63
_HEPW = _CH * _NCHH
_EPWP = _HALVES * _HEPW


def _sc_spmm(m, src, dst, w):
    N, H = m.shape
    RPS = 624
    TAIL = N - _NS * RPS
    TBASE = _NS * RPS

    mesh = plsc.VectorSubcoreMesh(
        core_axis_name="c", subcore_axis_name="s",
        num_cores=_NC, num_subcores=_NS)

    @functools.partial(
        pl.kernel,
        out_type=jax.ShapeDtypeStruct((_NC, N, H), jnp.float32),
        mesh=mesh,
        scratch_types=[
            pltpu.VMEM((_HEPW,), jnp.int32),
            pltpu.VMEM((_HEPW,), jnp.int32),
            pltpu.VMEM((_HEPW,), jnp.float32),
            [pltpu.VMEM((_CH, H), jnp.float32) for _ in range(3)],
            [pltpu.VMEM((_CH,), jnp.int32) for _ in range(3)],
            pltpu.VMEM_SHARED((N, H), jnp.float32),
            [pltpu.SemaphoreType.DMA for _ in range(3)],
            [pltpu.SemaphoreType.DMA for _ in range(3)],
        ],
    )
    def k(m_hbm, src_hbm, dst_hbm, w_hbm, out_hbm,
          src_b, dst_b, w_b, rows, dst_v, agg_sh, gsem, ssem):
        cid = lax.axis_index("c")
        sid = lax.axis_index("s")
        wid = cid * _NS + sid

        zero16 = jnp.zeros((16,), jnp.float32)

        def zrow(i, carry):
            for j in range(H // 16):
                rows[0][i, pl.ds(j * 16, 16)] = zero16
            return carry
        lax.fori_loop(0, _CH, zrow, 0)
        rbase = sid * RPS
        REM = RPS % _CH

        for t in range(RPS // _CH):
            pltpu.sync_copy(rows[0], agg_sh.at[pl.ds(rbase + t * _CH, _CH)])
        if REM:
            pltpu.sync_copy(rows[0].at[pl.ds(0, REM)],
                            agg_sh.at[pl.ds(rbase + RPS - REM, REM)])

        @pl.when(sid == _NS - 1)
        def _zero_tail():
            pltpu.sync_copy(rows[0].at[pl.ds(0, TAIL)],
                            agg_sh.at[pl.ds(TBASE, TAIL)])
        plsc.subcore_barrier()

        def start_gather(c, k_):
            pltpu.async_copy(
                m_hbm.at[src_b.at[pl.ds(c * _CH, _CH)]], rows[k_], gsem[k_])

        def wait_gather(k_):
            pltpu.make_async_copy(
                m_hbm.at[src_b.at[pl.ds(0, _CH)]], rows[k_], gsem[k_]).wait()

        def start_scatter(k_):
            pltpu.async_copy(rows[k_], agg_sh.at[dst_v[k_]], ssem[k_],
                             add=True)

        def wait_scatter(k_):
            pltpu.make_async_copy(rows[k_], agg_sh.at[dst_v[k_]],
                                  ssem[k_]).wait()

        for h in range(_HALVES):
            base = wid * _EPWP + h * _HEPW
            pltpu.sync_copy(src_hbm.at[pl.ds(base, _HEPW)], src_b)
            pltpu.sync_copy(dst_hbm.at[pl.ds(base, _HEPW)], dst_b)
            pltpu.sync_copy(w_hbm.at[pl.ds(base, _HEPW)], w_b)
            start_gather(0, 0)

            def triple(t, carry):
                for k_ in range(3):
                    c = 3 * t + k_
                    kn = (k_ + 1) % 3

                    @pl.when(c >= 2)
                    def _w():
                        wait_scatter(kn)

                    @pl.when(c + 1 < _NCHH)
                    def _g():
                        start_gather(c + 1, kn)
                    wait_gather(k_)
                    for g in range(_CH // 16):
                        dst_v[k_][pl.ds(g * 16, 16)] = (
                            dst_b[pl.ds(c * _CH + g * 16, 16)])

                    def scale(g, c2):
                        w16 = w_b[pl.ds(c * _CH + g * 16, 16)]
                        for r in range(16):
                            wv = jnp.full((16,), w16[r])
                            for j in range(H // 16):
                                rows[k_][g * 16 + r,
                                         pl.ds(j * 16, 16)] = (
                                    rows[k_][g * 16 + r,
                                             pl.ds(j * 16, 16)] * wv)
                        return c2
                    lax.fori_loop(0, _CH // 16, scale, 0)
                    start_scatter(k_)
                return carry
            lax.fori_loop(0, _NCHH // 3, triple, 0)
            wait_scatter((_NCHH - 2) % 3)
            wait_scatter((_NCHH - 1) % 3)

        plsc.subcore_barrier()
        for t in range(RPS // _CH):
            pltpu.sync_copy(agg_sh.at[pl.ds(rbase + t * _CH, _CH)],
                            rows[1])
            pltpu.sync_copy(
                rows[1], out_hbm.at[cid, pl.ds(rbase + t * _CH, _CH)])
        if REM:
            pltpu.sync_copy(agg_sh.at[pl.ds(rbase + RPS - REM, REM)],
                            rows[1].at[pl.ds(0, REM)])
            pltpu.sync_copy(
                rows[1].at[pl.ds(0, REM)],
                out_hbm.at[cid, pl.ds(rbase + RPS - REM, REM)])

        @pl.when(sid == _NS - 1)
        def _drain_tail():
            pltpu.sync_copy(agg_sh.at[pl.ds(TBASE, TAIL)],
                            rows[1].at[pl.ds(0, TAIL)])
            pltpu.sync_copy(rows[1].at[pl.ds(0, TAIL)],
                            out_hbm.at[cid, pl.ds(TBASE, TAIL)])

    return k(m, src, dst, w)


_BN = 1000


def _embed_body(h_ref, WeT_ref, be_ref, Wg0_ref, x_ref, m0_ref):
    x = jnp.dot(h_ref[...], WeT_ref[...],
                preferred_element_type=jnp.float32) + be_ref[...]
    x_ref[...] = x
    m0_ref[...] = jnp.dot(x, Wg0_ref[...], preferred_element_type=jnp.float32)


def _embed(h, WeT, be2, Wg0):
    N, D = h.shape
    H = WeT.shape[1]
    grid = (N // _BN,)
    return pl.pallas_call(
        _embed_body,
        grid=grid,
        in_specs=[
            pl.BlockSpec((_BN, D), lambda i: (i, 0)),
            pl.BlockSpec((D, H), lambda i: (0, 0)),
            pl.BlockSpec((1, H), lambda i: (0, 0)),
            pl.BlockSpec((H, H), lambda i: (0, 0)),
        ],
        out_specs=[
            pl.BlockSpec((_BN, H), lambda i: (i, 0)),
            pl.BlockSpec((_BN, H), lambda i: (i, 0)),
        ],
        out_shape=[
            jax.ShapeDtypeStruct((N, H), jnp.float32),
            jax.ShapeDtypeStruct((N, H), jnp.float32),
        ],
    )(h, WeT, be2, Wg0)


def _gru_body(agg2_ref, x_ref, WihT_ref, WhhT_ref, bih_ref, bhh_ref, Wgn_ref,
              xn_ref, mn_ref):
    H = x_ref.shape[1]
    agg = agg2_ref[0] + agg2_ref[1]
    x = x_ref[...]
    gi = jnp.dot(agg, WihT_ref[...],
                 preferred_element_type=jnp.float32) + bih_ref[...]
    gh = jnp.dot(x, WhhT_ref[...],
                 preferred_element_type=jnp.float32) + bhh_ref[...]
    r = jax.nn.sigmoid(gi[:, :H] + gh[:, :H])
    z = jax.nn.sigmoid(gi[:, H:2 * H] + gh[:, H:2 * H])
    n = jnp.tanh(gi[:, 2 * H:] + r * gh[:, 2 * H:])
    xn = (1.0 - z) * n + z * x
    xn_ref[...] = xn
    mn_ref[...] = jnp.dot(xn, Wgn_ref[...],
                          preferred_element_type=jnp.float32)


def _gru(agg2, x, WihT, WhhT, bih2, bhh2, Wgn):
    N, H = x.shape
    grid = (N // _BN,)
    return pl.pallas_call(
        _gru_body,
        grid=grid,
        in_specs=[
            pl.BlockSpec((_NC, _BN, H), lambda i: (0, i, 0)),
            pl.BlockSpec((_BN, H), lambda i: (i, 0)),
            pl.BlockSpec((H, 3 * H), lambda i: (0, 0)),
            pl.BlockSpec((H, 3 * H), lambda i: (0, 0)),
            pl.BlockSpec((1, 3 * H), lambda i: (0, 0)),
            pl.BlockSpec((1, 3 * H), lambda i: (0, 0)),
            pl.BlockSpec((H, H), lambda i: (0, 0)),
        ],
        out_specs=[
            pl.BlockSpec((_BN, H), lambda i: (i, 0)),
            pl.BlockSpec((_BN, H), lambda i: (i, 0)),
        ],
        out_shape=[
            jax.ShapeDtypeStruct((N, H), jnp.float32),
            jax.ShapeDtypeStruct((N, H), jnp.float32),
        ],
    )(agg2, x, WihT, WhhT, bih2, bhh2, Wgn)


def _mlp_body(xin_ref, x_ref, WmT_ref, bm_ref, out_ref):
    s = xin_ref[...] + x_ref[...]
    out_ref[...] = jnp.dot(s, WmT_ref[...],
                           preferred_element_type=jnp.float32) + bm_ref[...]


def _mlp(x_in, x, WmT_pad, bm_pad):
    N, H = x.shape
    CP = WmT_pad.shape[1]
    grid = (N // _BN,)
    return pl.pallas_call(
        _mlp_body,
        grid=grid,
        in_specs=[
            pl.BlockSpec((_BN, H), lambda i: (i, 0)),
            pl.BlockSpec((_BN, H), lambda i: (i, 0)),
            pl.BlockSpec((H, CP), lambda i: (0, 0)),
            pl.BlockSpec((1, CP), lambda i: (0, 0)),
        ],
        out_specs=pl.BlockSpec((_BN, CP), lambda i: (i, 0)),
        out_shape=jax.ShapeDtypeStruct((N, CP), jnp.float32),
    )(x_in, x, WmT_pad, bm_pad)


def kernel(h, edge_index, edge_weight, We, be, Wg, W_ih, W_hh, b_ih, b_hh,
           Wm, bm):
    L, H, _ = Wg.shape
    C = Wm.shape[0]
    src = edge_index[0].astype(jnp.int32)
    dst = edge_index[1].astype(jnp.int32)
    w = edge_weight.astype(jnp.float32)
    E = src.shape[0]
    EPW = E // _NW
    PAD = _EPWP - EPW
    src = jnp.pad(src.reshape(_NW, EPW), ((0, 0), (0, PAD))).reshape(-1)
    dst = jnp.pad(dst.reshape(_NW, EPW), ((0, 0), (0, PAD))).reshape(-1)
    w = jnp.pad(w.reshape(_NW, EPW), ((0, 0), (0, PAD))).reshape(-1)

    WeT = We.T
    be2 = be.reshape(1, H)
    WihT = W_ih.T
    WhhT = W_hh.T
    bih2 = b_ih.reshape(1, 3 * H)
    bhh2 = b_hh.reshape(1, 3 * H)
    CP = 128
    WmT_pad = jnp.zeros((H, CP), jnp.float32).at[:, :C].set(Wm.T)
    bm_pad = jnp.zeros((1, CP), jnp.float32).at[:, :C].set(bm.reshape(1, C))

    x, m = _embed(h, WeT, be2, Wg[0])
    x_in = x
    for i in range(L):
        agg2 = _sc_spmm(m, src, dst, w)
        Wgn = Wg[i + 1] if i + 1 < L else Wg[0]
        x, m = _gru(agg2, x, WihT, WhhT, bih2, bhh2, Wgn)
    out = _mlp(x_in, x, WmT_pad, bm_pad)
    return out[:, :C]
